# baseline (speedup 1.0000x reference)
import functools

import jax
import jax.numpy as jnp
from jax import lax
from jax.experimental import pallas as pl
from jax.experimental.pallas import tpu as pltpu
from jax.experimental.pallas import tpu_sc as plsc

_B = 16384
_V = 1000000
_D = 64
_H = 128
_B4 = 4 * _B

_NC = 2
_NS = 16
_NW = _NC * _NS
_ROWS_PER_W = _B4 // _NW
_CHUNK = 128
_NCHUNK = _ROWS_PER_W // _CHUNK
_MEGA = 2
_NSTAGE = _NCHUNK // _MEGA
_SROWS = _MEGA * _CHUNK


def _sc_gather_body(table_hbm, idx_hbm, out_hbm, idx_v, rows_v, sem, out_sem):
    wid = lax.axis_index("s") * _NC + lax.axis_index("c")
    base = wid * _ROWS_PER_W
    pltpu.sync_copy(idx_hbm.at[wid], idx_v)
    out_copies = [None, None]
    for m in range(_NSTAGE):
        b = m % 2
        if out_copies[b] is not None:
            out_copies[b].wait()
        gathers = [
            pltpu.async_copy(table_hbm.at[idx_v.at[m * _MEGA + j]],
                             rows_v.at[b, pl.ds(j * _CHUNK, _CHUNK)], sem)
            for j in range(_MEGA)
        ]
        for g in gathers:
            g.wait()
        out_copies[b] = pltpu.async_copy(
            rows_v.at[b], out_hbm.at[pl.ds(base + m * _SROWS, _SROWS)],
            out_sem)
    for cp in out_copies:
        cp.wait()


@functools.lru_cache(maxsize=1)
def _make_sc_gather():
    return pl.kernel(
        _sc_gather_body,
        out_type=jax.ShapeDtypeStruct((_B4, 2 * _D), jnp.int32),
        mesh=plsc.VectorSubcoreMesh(core_axis_name="c", subcore_axis_name="s"),
        scratch_types=[
            pltpu.VMEM((_NCHUNK, _CHUNK), jnp.int32),
            pltpu.VMEM((2, _SROWS, 2 * _D), jnp.int32),
            pltpu.SemaphoreType.DMA,
            pltpu.SemaphoreType.DMA,
        ],
    )


def _sc_gather(table2, idx):
    return _make_sc_gather()(table2, idx.reshape(_NW, _NCHUNK, _CHUNK))


_HOFF4 = 249984
_R4 = _V - 3 * _HOFF4
_TBLK = 8064
_TSTEPS = 31
_TGRID = 32


def _pack_words(t):
    tb = t.astype(jnp.bfloat16)
    lo = lax.bitcast_convert_type(tb[:, :32], jnp.uint16).astype(jnp.int32)
    hi = lax.bitcast_convert_type(tb[:, 32:], jnp.uint16).astype(jnp.int32)
    return lo | (hi << 16)


def _transpose_body(xa_ref, xb_ref, xc_ref, xd_ref, eye_ref, out_ref):
    ta = xa_ref[...].T
    tc = xc_ref[...].T
    dims = (((0,), (0,)), ((), ()))
    tb = lax.dot_general(xb_ref[...], eye_ref[...], dims,
                         preferred_element_type=jnp.float32)
    td = lax.dot_general(xd_ref[...], eye_ref[...], dims,
                         preferred_element_type=jnp.float32)
    out_ref[...] = jnp.concatenate(
        [_pack_words(ta), _pack_words(tb),
         _pack_words(tc), _pack_words(td)], axis=1)


def _tc_transpose(table_t):
    eye = jnp.eye(_D, dtype=jnp.float32)
    return pl.pallas_call(
        _transpose_body,
        grid=(_TGRID,),
        in_specs=[
            pl.BlockSpec((_D, _TBLK), lambda i: (0, i)),
            pl.BlockSpec((_D, _TBLK), lambda i: (0, i + _TSTEPS)),
            pl.BlockSpec((_D, _TBLK), lambda i: (0, i + 2 * _TSTEPS)),
            pl.BlockSpec((_D, _TBLK), lambda i: (0, i + 3 * _TSTEPS)),
            pl.BlockSpec((_D, _D), lambda i: (0, 0)),
        ],
        out_specs=pl.BlockSpec((_TBLK, 2 * _D), lambda i: (i, 0)),
        out_shape=jax.ShapeDtypeStruct((_R4, 2 * _D), jnp.int32),
    )(table_t, table_t, table_t, table_t, eye)


_BLK = 2048


def _mlp_body(g_ref, p_ref, s_ref, w1a_ref, w1s_ref, b1_ref, w2_ref, b2_ref,
              out_ref):
    parts = []
    for j in range(4):
        base = j * 2 * _D
        qj = p_ref[:, j][:, None]
        quarters = [g_ref[:, base + 32 * q: base + 32 * (q + 1)]
                    for q in range(4)]
        w = jnp.where(qj < 0.5, quarters[0],
                      jnp.where(qj < 1.5, quarters[1],
                                jnp.where(qj < 2.5, quarters[2],
                                          quarters[3])))
        lo = lax.bitcast_convert_type(
            (w & 0xFFFF).astype(jnp.uint16), jnp.bfloat16)
        hi = lax.bitcast_convert_type(
            (w >> 16).astype(jnp.uint16), jnp.bfloat16)
        parts.append(jnp.concatenate([lo, hi], axis=1).astype(jnp.float32))
    x = jnp.concatenate(parts, axis=1)
    h = jnp.dot(x, w1a_ref[...], preferred_element_type=jnp.float32)
    h = h + jnp.dot(s_ref[...], w1s_ref[...],
                    preferred_element_type=jnp.float32)
    h = jnp.maximum(h + b1_ref[...][None, :], 0.0)
    out_ref[...] = jnp.sum(h * w2_ref[...][None, :], axis=1) + b2_ref[...]


def _tc_mlp(g, p, s, w1a, w1s, b1, w2row, b2):
    grid = _B // _BLK
    return pl.pallas_call(
        _mlp_body,
        grid=(grid,),
        in_specs=[
            pl.BlockSpec((_BLK, 8 * _D), lambda i: (i, 0)),
            pl.BlockSpec((_BLK, 8), lambda i: (i, 0)),
            pl.BlockSpec((_BLK, 8), lambda i: (i, 0)),
            pl.BlockSpec((4 * _D, _H), lambda i: (0, 0)),
            pl.BlockSpec((8, _H), lambda i: (0, 0)),
            pl.BlockSpec((_H,), lambda i: (0,)),
            pl.BlockSpec((_H,), lambda i: (0,)),
            pl.BlockSpec((1,), lambda i: (0,)),
        ],
        out_specs=pl.BlockSpec((_BLK,), lambda i: (i,)),
        out_shape=jax.ShapeDtypeStruct((_B,), jnp.float32),
    )(g, p, s, w1a, w1s, b1, w2row, b2)


def kernel(target_item_ids, original_item_ids, left_item_ids, right_item_ids,
           position_indices, normalized_positions, session_lengths,
           prefix_scores, has_prefixes, item_embedding, W1, b1, W2, b2):
    ids = jnp.stack([target_item_ids, original_item_ids,
                     left_item_ids, right_item_ids], axis=1)
    q = jnp.minimum(ids // _HOFF4, 3)
    idx = (ids - q * _HOFF4).reshape(_B4)
    parity = q

    table2 = _tc_transpose(item_embedding.T)
    gathered = _sc_gather(table2, idx)
    g = gathered.reshape(_B, 8 * _D)

    p = jnp.concatenate([parity.astype(jnp.float32),
                         jnp.zeros((_B, 4), jnp.float32)], axis=1)
    zeros = jnp.zeros((_B,), jnp.float32)
    s = jnp.stack([position_indices, normalized_positions, session_lengths,
                   prefix_scores, has_prefixes, zeros, zeros, zeros], axis=1)

    w1a = W1[: 4 * _D]
    w1s = jnp.zeros((8, _H), jnp.float32).at[:5].set(W1[4 * _D:])
    w2row = W2.reshape(_H)
    return _tc_mlp(g, p, s, w1a, w1s, b1, w2row, b2)


# speedup vs baseline: 1.2492x; 1.2492x over previous
import functools

import jax
import jax.numpy as jnp
from jax import lax
from jax.experimental import pallas as pl
from jax.experimental.pallas import tpu as pltpu
from jax.experimental.pallas import tpu_sc as plsc

_B = 16384
_V = 1000000
_D = 64
_H = 128
_B4 = 4 * _B

_NC = 2
_NS = 16
_NW = _NC * _NS
_ROWS_PER_W = _B4 // _NW
_CHUNK = 128
_NCHUNK = _ROWS_PER_W // _CHUNK
_MEGA = 2
_NSTAGE = _NCHUNK // _MEGA
_SROWS = _MEGA * _CHUNK


def _sc_gather_body(table_hbm, idx_hbm, out_hbm, idx_v, rows_v, sem, out_sem):
    wid = lax.axis_index("s") * _NC + lax.axis_index("c")
    base = wid * _ROWS_PER_W
    pltpu.sync_copy(idx_hbm.at[wid], idx_v)
    out_copies = [None, None]
    for m in range(_NSTAGE):
        b = m % 2
        if out_copies[b] is not None:
            out_copies[b].wait()
        gathers = [
            pltpu.async_copy(table_hbm.at[idx_v.at[m * _MEGA + j]],
                             rows_v.at[b, pl.ds(j * _CHUNK, _CHUNK)], sem)
            for j in range(_MEGA)
        ]
        for g in gathers:
            g.wait()
        out_copies[b] = pltpu.async_copy(
            rows_v.at[b], out_hbm.at[pl.ds(base + m * _SROWS, _SROWS)],
            out_sem)
    for cp in out_copies:
        cp.wait()


@functools.lru_cache(maxsize=1)
def _make_sc_gather():
    return pl.kernel(
        _sc_gather_body,
        out_type=jax.ShapeDtypeStruct((_B4, 2 * _D), jnp.float32),
        mesh=plsc.VectorSubcoreMesh(core_axis_name="c", subcore_axis_name="s"),
        scratch_types=[
            pltpu.VMEM((_NCHUNK, _CHUNK), jnp.int32),
            pltpu.VMEM((2, _SROWS, 2 * _D), jnp.float32),
            pltpu.SemaphoreType.DMA,
            pltpu.SemaphoreType.DMA,
        ],
    )


def _sc_gather(table2, idx):
    return _make_sc_gather()(table2, idx.reshape(_NW, _NCHUNK, _CHUNK))


_HOFF = 499968
_R = _V - _HOFF
_TBLK = 5376
_TGRID = 94


def _transpose_body(xa_ref, xb_ref, eye_ref, out_ref):
    ta = xa_ref[...].T
    tb = lax.dot_general(xb_ref[...], eye_ref[...],
                         (((0,), (0,)), ((), ())),
                         preferred_element_type=jnp.float32)
    out_ref[...] = jnp.concatenate([ta, tb], axis=1)


def _tc_transpose(table_t):
    eye = jnp.eye(_D, dtype=jnp.float32)
    return pl.pallas_call(
        _transpose_body,
        grid=(_TGRID,),
        in_specs=[
            pl.BlockSpec((_D, _TBLK), lambda i: (0, i)),
            pl.BlockSpec((_D, _TBLK), lambda i: (0, i + _TGRID - 1)),
            pl.BlockSpec((_D, _D), lambda i: (0, 0)),
        ],
        out_specs=pl.BlockSpec((_TBLK, 2 * _D), lambda i: (i, 0)),
        out_shape=jax.ShapeDtypeStruct((_R, 2 * _D), jnp.float32),
    )(table_t, table_t, eye)


_BLK = 2048


def _mlp_body(g_ref, p_ref, s_ref, w1a_ref, w1s_ref, b1_ref, w2_ref, b2_ref,
              out_ref):
    parts = []
    for j in range(4):
        lo = g_ref[:, j * 2 * _D: j * 2 * _D + _D]
        hi = g_ref[:, j * 2 * _D + _D: (j + 1) * 2 * _D]
        pj = p_ref[:, j][:, None]
        parts.append(jnp.where(pj > 0.5, hi, lo))
    x = jnp.concatenate(parts, axis=1)
    h = jnp.dot(x, w1a_ref[...], preferred_element_type=jnp.float32)
    h = h + jnp.dot(s_ref[...], w1s_ref[...],
                    preferred_element_type=jnp.float32)
    h = jnp.maximum(h + b1_ref[...][None, :], 0.0)
    out_ref[...] = jnp.sum(h * w2_ref[...][None, :], axis=1) + b2_ref[...]


def _tc_mlp(g, p, s, w1a, w1s, b1, w2row, b2):
    grid = _B // _BLK
    return pl.pallas_call(
        _mlp_body,
        grid=(grid,),
        in_specs=[
            pl.BlockSpec((_BLK, 8 * _D), lambda i: (i, 0)),
            pl.BlockSpec((_BLK, 8), lambda i: (i, 0)),
            pl.BlockSpec((_BLK, 8), lambda i: (i, 0)),
            pl.BlockSpec((4 * _D, _H), lambda i: (0, 0)),
            pl.BlockSpec((8, _H), lambda i: (0, 0)),
            pl.BlockSpec((_H,), lambda i: (0,)),
            pl.BlockSpec((_H,), lambda i: (0,)),
            pl.BlockSpec((1,), lambda i: (0,)),
        ],
        out_specs=pl.BlockSpec((_BLK,), lambda i: (i,)),
        out_shape=jax.ShapeDtypeStruct((_B,), jnp.float32),
    )(g, p, s, w1a, w1s, b1, w2row, b2)


def kernel(target_item_ids, original_item_ids, left_item_ids, right_item_ids,
           position_indices, normalized_positions, session_lengths,
           prefix_scores, has_prefixes, item_embedding, W1, b1, W2, b2):
    ids = jnp.stack([target_item_ids, original_item_ids,
                     left_item_ids, right_item_ids], axis=1)
    half = ids >= _HOFF
    idx = jnp.where(half, ids - _HOFF, ids).reshape(_B4)
    parity = half.astype(jnp.int32)

    table2 = _tc_transpose(item_embedding.T)
    gathered = _sc_gather(table2, idx)
    g = gathered.reshape(_B, 8 * _D)

    p = jnp.concatenate([parity.astype(jnp.float32),
                         jnp.zeros((_B, 4), jnp.float32)], axis=1)
    zeros = jnp.zeros((_B,), jnp.float32)
    s = jnp.stack([position_indices, normalized_positions, session_lengths,
                   prefix_scores, has_prefixes, zeros, zeros, zeros], axis=1)

    w1a = W1[: 4 * _D]
    w1s = jnp.zeros((8, _H), jnp.float32).at[:5].set(W1[4 * _D:])
    w2row = W2.reshape(_H)
    return _tc_mlp(g, p, s, w1a, w1s, b1, w2row, b2)


# speedup vs baseline: 1.3272x; 1.0624x over previous
import functools

import jax
import jax.numpy as jnp
from jax import lax
from jax.experimental import pallas as pl
from jax.experimental.pallas import tpu as pltpu
from jax.experimental.pallas import tpu_sc as plsc

_B = 16384
_V = 1000000
_D = 64
_H = 128
_B4 = 4 * _B

_NC = 2
_NS = 16
_NW = _NC * _NS
_ROWS_PER_W = _B4 // _NW
_CHUNK = 128
_NCHUNK = _ROWS_PER_W // _CHUNK
_MEGA = 2
_NSTAGE = _NCHUNK // _MEGA
_SROWS = _MEGA * _CHUNK


def _sc_gather_body(table_hbm, idx_hbm, out_hbm, idx_v, rows_v, sem, out_sem):
    wid = lax.axis_index("s") * _NC + lax.axis_index("c")
    base = wid * _ROWS_PER_W
    pltpu.sync_copy(idx_hbm.at[wid], idx_v)
    out_copies = [None, None]
    for m in range(_NSTAGE):
        b = m % 2
        if out_copies[b] is not None:
            out_copies[b].wait()
        gathers = [
            pltpu.async_copy(table_hbm.at[idx_v.at[m * _MEGA + j]],
                             rows_v.at[b, pl.ds(j * _CHUNK, _CHUNK)], sem)
            for j in range(_MEGA)
        ]
        for g in gathers:
            g.wait()
        out_copies[b] = pltpu.async_copy(
            rows_v.at[b], out_hbm.at[pl.ds(base + m * _SROWS, _SROWS)],
            out_sem)
    for cp in out_copies:
        cp.wait()


@functools.lru_cache(maxsize=1)
def _make_sc_gather():
    return pl.kernel(
        _sc_gather_body,
        out_type=jax.ShapeDtypeStruct((_B4, 2 * _D), jnp.float32),
        mesh=plsc.VectorSubcoreMesh(core_axis_name="c", subcore_axis_name="s"),
        scratch_types=[
            pltpu.VMEM((_NCHUNK, _CHUNK), jnp.int32),
            pltpu.VMEM((2, _SROWS, 2 * _D), jnp.float32),
            pltpu.SemaphoreType.DMA,
            pltpu.SemaphoreType.DMA,
        ],
    )


def _sc_gather(table2, idx):
    return _make_sc_gather()(table2, idx.reshape(_NW, _NCHUNK, _CHUNK))


_HOFF = 499968
_R = _V - _HOFF
_TBLK = 5376
_TGRID = 94


_XLU_LANES = 1792


def _transpose_body(xa_ref, xb_ref, eye_ref, out_ref):
    def mxu_t(z):
        return lax.dot_general(z.astype(jnp.bfloat16), eye_ref[...],
                               (((0,), (0,)), ((), ())),
                               preferred_element_type=jnp.float32)

    ta = jnp.concatenate([xa_ref[:, :_XLU_LANES].T,
                          mxu_t(xa_ref[:, _XLU_LANES:])], axis=0)
    tb = mxu_t(xb_ref[...])
    out_ref[...] = jnp.concatenate([ta, tb], axis=1)


def _tc_transpose(table_t):
    eye = jnp.eye(_D, dtype=jnp.bfloat16)
    return pl.pallas_call(
        _transpose_body,
        grid=(_TGRID,),
        in_specs=[
            pl.BlockSpec((_D, _TBLK), lambda i: (0, i)),
            pl.BlockSpec((_D, _TBLK), lambda i: (0, i + _TGRID - 1)),
            pl.BlockSpec((_D, _D), lambda i: (0, 0)),
        ],
        out_specs=pl.BlockSpec((_TBLK, 2 * _D), lambda i: (i, 0)),
        out_shape=jax.ShapeDtypeStruct((_R, 2 * _D), jnp.float32),
    )(table_t, table_t, eye)


_BLK = 2048


def _mlp_body(g_ref, p_ref, s_ref, w1a_ref, w1s_ref, b1_ref, w2_ref, b2_ref,
              out_ref):
    parts = []
    for j in range(4):
        lo = g_ref[:, j * 2 * _D: j * 2 * _D + _D]
        hi = g_ref[:, j * 2 * _D + _D: (j + 1) * 2 * _D]
        pj = p_ref[:, j][:, None]
        parts.append(jnp.where(pj > 0.5, hi, lo))
    x = jnp.concatenate(parts, axis=1)
    h = jnp.dot(x, w1a_ref[...], preferred_element_type=jnp.float32)
    h = h + jnp.dot(s_ref[...], w1s_ref[...],
                    preferred_element_type=jnp.float32)
    h = jnp.maximum(h + b1_ref[...][None, :], 0.0)
    out_ref[...] = jnp.sum(h * w2_ref[...][None, :], axis=1) + b2_ref[...]


def _tc_mlp(g, p, s, w1a, w1s, b1, w2row, b2):
    grid = _B // _BLK
    return pl.pallas_call(
        _mlp_body,
        grid=(grid,),
        in_specs=[
            pl.BlockSpec((_BLK, 8 * _D), lambda i: (i, 0)),
            pl.BlockSpec((_BLK, 8), lambda i: (i, 0)),
            pl.BlockSpec((_BLK, 8), lambda i: (i, 0)),
            pl.BlockSpec((4 * _D, _H), lambda i: (0, 0)),
            pl.BlockSpec((8, _H), lambda i: (0, 0)),
            pl.BlockSpec((_H,), lambda i: (0,)),
            pl.BlockSpec((_H,), lambda i: (0,)),
            pl.BlockSpec((1,), lambda i: (0,)),
        ],
        out_specs=pl.BlockSpec((_BLK,), lambda i: (i,)),
        out_shape=jax.ShapeDtypeStruct((_B,), jnp.float32),
    )(g, p, s, w1a, w1s, b1, w2row, b2)


def kernel(target_item_ids, original_item_ids, left_item_ids, right_item_ids,
           position_indices, normalized_positions, session_lengths,
           prefix_scores, has_prefixes, item_embedding, W1, b1, W2, b2):
    ids = jnp.stack([target_item_ids, original_item_ids,
                     left_item_ids, right_item_ids], axis=1)
    half = ids >= _HOFF
    idx = jnp.where(half, ids - _HOFF, ids).reshape(_B4)
    parity = half.astype(jnp.int32)

    table2 = _tc_transpose(item_embedding.T)
    gathered = _sc_gather(table2, idx)
    g = gathered.reshape(_B, 8 * _D)

    p = jnp.concatenate([parity.astype(jnp.float32),
                         jnp.zeros((_B, 4), jnp.float32)], axis=1)
    zeros = jnp.zeros((_B,), jnp.float32)
    s = jnp.stack([position_indices, normalized_positions, session_lengths,
                   prefix_scores, has_prefixes, zeros, zeros, zeros], axis=1)

    w1a = W1[: 4 * _D]
    w1s = jnp.zeros((8, _H), jnp.float32).at[:5].set(W1[4 * _D:])
    w2row = W2.reshape(_H)
    return _tc_mlp(g, p, s, w1a, w1s, b1, w2row, b2)


# speedup vs baseline: 1.3292x; 1.0015x over previous
import functools

import jax
import jax.numpy as jnp
from jax import lax
from jax.experimental import pallas as pl
from jax.experimental.pallas import tpu as pltpu
from jax.experimental.pallas import tpu_sc as plsc

_B = 16384
_V = 1000000
_D = 64
_H = 128
_B4 = 4 * _B

_NC = 2
_NS = 16
_NW = _NC * _NS
_ROWS_PER_W = _B4 // _NW
_CHUNK = 128
_NCHUNK = _ROWS_PER_W // _CHUNK
_MEGA = 2
_NSTAGE = _NCHUNK // _MEGA
_SROWS = _MEGA * _CHUNK


def _sc_gather_body(table_hbm, idx_hbm, out_hbm, idx_v, rows_v, sem, out_sem):
    wid = lax.axis_index("s") * _NC + lax.axis_index("c")
    base = wid * _ROWS_PER_W
    pltpu.sync_copy(idx_hbm.at[wid], idx_v)
    out_copies = [None, None]
    for m in range(_NSTAGE):
        b = m % 2
        if out_copies[b] is not None:
            out_copies[b].wait()
        gathers = [
            pltpu.async_copy(table_hbm.at[idx_v.at[m * _MEGA + j]],
                             rows_v.at[b, pl.ds(j * _CHUNK, _CHUNK)], sem)
            for j in range(_MEGA)
        ]
        for g in gathers:
            g.wait()
        out_copies[b] = pltpu.async_copy(
            rows_v.at[b], out_hbm.at[pl.ds(base + m * _SROWS, _SROWS)],
            out_sem)
    for cp in out_copies:
        cp.wait()


@functools.lru_cache(maxsize=1)
def _make_sc_gather():
    return pl.kernel(
        _sc_gather_body,
        out_type=jax.ShapeDtypeStruct((_B4, 2 * _D), jnp.float32),
        mesh=plsc.VectorSubcoreMesh(core_axis_name="c", subcore_axis_name="s"),
        scratch_types=[
            pltpu.VMEM((_NCHUNK, _CHUNK), jnp.int32),
            pltpu.VMEM((2, _SROWS, 2 * _D), jnp.float32),
            pltpu.SemaphoreType.DMA,
            pltpu.SemaphoreType.DMA,
        ],
    )


def _sc_gather(table2, idx):
    return _make_sc_gather()(table2, idx.reshape(_NW, _NCHUNK, _CHUNK))


_HOFF = 499968
_R = _V - _HOFF
_TBLK = 5376
_TGRID = 94


_XLU_LANES = 1536


def _transpose_body(xa_ref, xb_ref, eye_ref, out_ref):
    def mxu_t(z):
        return lax.dot_general(z.astype(jnp.bfloat16), eye_ref[...],
                               (((0,), (0,)), ((), ())),
                               preferred_element_type=jnp.float32)

    ta = jnp.concatenate([xa_ref[:, :_XLU_LANES].T,
                          mxu_t(xa_ref[:, _XLU_LANES:])], axis=0)
    tb = mxu_t(xb_ref[...])
    out_ref[...] = jnp.concatenate([ta, tb], axis=1)


def _tc_transpose(table_t):
    eye = jnp.eye(_D, dtype=jnp.bfloat16)
    return pl.pallas_call(
        _transpose_body,
        grid=(_TGRID,),
        in_specs=[
            pl.BlockSpec((_D, _TBLK), lambda i: (0, i)),
            pl.BlockSpec((_D, _TBLK), lambda i: (0, i + _TGRID - 1)),
            pl.BlockSpec((_D, _D), lambda i: (0, 0)),
        ],
        out_specs=pl.BlockSpec((_TBLK, 2 * _D), lambda i: (i, 0)),
        out_shape=jax.ShapeDtypeStruct((_R, 2 * _D), jnp.float32),
    )(table_t, table_t, eye)


_BLK = 2048


def _mlp_body(g_ref, p_ref, s_ref, w1a_ref, w1s_ref, b1_ref, w2_ref, b2_ref,
              out_ref):
    parts = []
    for j in range(4):
        lo = g_ref[:, j * 2 * _D: j * 2 * _D + _D]
        hi = g_ref[:, j * 2 * _D + _D: (j + 1) * 2 * _D]
        pj = p_ref[:, j][:, None]
        parts.append(jnp.where(pj > 0.5, hi, lo))
    x = jnp.concatenate(parts, axis=1).astype(jnp.bfloat16)
    h = jnp.dot(x, w1a_ref[...].astype(jnp.bfloat16),
                preferred_element_type=jnp.float32)
    h = h + jnp.dot(s_ref[...], w1s_ref[...],
                    preferred_element_type=jnp.float32)
    h = jnp.maximum(h + b1_ref[...][None, :], 0.0)
    out_ref[...] = jnp.sum(h * w2_ref[...][None, :], axis=1) + b2_ref[...]


def _tc_mlp(g, p, s, w1a, w1s, b1, w2row, b2):
    grid = _B // _BLK
    return pl.pallas_call(
        _mlp_body,
        grid=(grid,),
        in_specs=[
            pl.BlockSpec((_BLK, 8 * _D), lambda i: (i, 0)),
            pl.BlockSpec((_BLK, 8), lambda i: (i, 0)),
            pl.BlockSpec((_BLK, 8), lambda i: (i, 0)),
            pl.BlockSpec((4 * _D, _H), lambda i: (0, 0)),
            pl.BlockSpec((8, _H), lambda i: (0, 0)),
            pl.BlockSpec((_H,), lambda i: (0,)),
            pl.BlockSpec((_H,), lambda i: (0,)),
            pl.BlockSpec((1,), lambda i: (0,)),
        ],
        out_specs=pl.BlockSpec((_BLK,), lambda i: (i,)),
        out_shape=jax.ShapeDtypeStruct((_B,), jnp.float32),
    )(g, p, s, w1a, w1s, b1, w2row, b2)


def kernel(target_item_ids, original_item_ids, left_item_ids, right_item_ids,
           position_indices, normalized_positions, session_lengths,
           prefix_scores, has_prefixes, item_embedding, W1, b1, W2, b2):
    ids = jnp.stack([target_item_ids, original_item_ids,
                     left_item_ids, right_item_ids], axis=1)
    half = ids >= _HOFF
    idx = jnp.where(half, ids - _HOFF, ids).reshape(_B4)
    parity = half.astype(jnp.int32)

    table2 = _tc_transpose(item_embedding.T)
    gathered = _sc_gather(table2, idx)
    g = gathered.reshape(_B, 8 * _D)

    p = jnp.concatenate([parity.astype(jnp.float32),
                         jnp.zeros((_B, 4), jnp.float32)], axis=1)
    zeros = jnp.zeros((_B,), jnp.float32)
    s = jnp.stack([position_indices, normalized_positions, session_lengths,
                   prefix_scores, has_prefixes, zeros, zeros, zeros], axis=1)

    w1a = W1[: 4 * _D]
    w1s = jnp.zeros((8, _H), jnp.float32).at[:5].set(W1[4 * _D:])
    w2row = W2.reshape(_H)
    return _tc_mlp(g, p, s, w1a, w1s, b1, w2row, b2)


# speedup vs baseline: 1.5053x; 1.1325x over previous
import functools

import jax
import jax.numpy as jnp
from jax import lax
from jax.experimental import pallas as pl
from jax.experimental.pallas import tpu as pltpu
from jax.experimental.pallas import tpu_sc as plsc

_B = 16384
_V = 1000000
_D = 64
_H = 128
_B4 = 4 * _B

_NC = 2
_NS = 16
_NW = _NC * _NS
_ROWS_PER_W = _B4 // _NW
_CHUNK = 128
_NCHUNK = _ROWS_PER_W // _CHUNK
_MEGA = 2
_NSTAGE = _NCHUNK // _MEGA
_SROWS = _MEGA * _CHUNK


def _sc_gather_body(table_hbm, idx_hbm, out_hbm, idx_v, rows_v, sem, out_sem):
    wid = lax.axis_index("s") * _NC + lax.axis_index("c")
    base = wid * _ROWS_PER_W
    pltpu.sync_copy(idx_hbm.at[wid], idx_v)
    out_copies = [None, None]
    for m in range(_NSTAGE):
        b = m % 2
        if out_copies[b] is not None:
            out_copies[b].wait()
        gathers = [
            pltpu.async_copy(table_hbm.at[idx_v.at[m * _MEGA + j]],
                             rows_v.at[b, pl.ds(j * _CHUNK, _CHUNK)], sem)
            for j in range(_MEGA)
        ]
        for g in gathers:
            g.wait()
        out_copies[b] = pltpu.async_copy(
            rows_v.at[b], out_hbm.at[pl.ds(base + m * _SROWS, _SROWS)],
            out_sem)
    for cp in out_copies:
        cp.wait()


@functools.lru_cache(maxsize=1)
def _make_sc_gather():
    return pl.kernel(
        _sc_gather_body,
        out_type=jax.ShapeDtypeStruct((_B4, 2 * _D), jnp.float32),
        mesh=plsc.VectorSubcoreMesh(core_axis_name="c", subcore_axis_name="s"),
        scratch_types=[
            pltpu.VMEM((_NCHUNK, _CHUNK), jnp.int32),
            pltpu.VMEM((2, _SROWS, 2 * _D), jnp.float32),
            pltpu.SemaphoreType.DMA,
            pltpu.SemaphoreType.DMA,
        ],
    )


def _sc_gather(table2, idx):
    return _make_sc_gather()(table2, idx.reshape(_NW, _NCHUNK, _CHUNK))


_HOFF = 499968
_R = _V - _HOFF
_TBLK = 16128
_TGRID = 32


_XLU_LANES = 4608


def _transpose_body(xa_ref, xb_ref, eye_ref, out_ref):
    def mxu_t(z):
        return lax.dot_general(z.astype(jnp.bfloat16), eye_ref[...],
                               (((0,), (0,)), ((), ())),
                               preferred_element_type=jnp.float32)

    ta = jnp.concatenate([xa_ref[:, :_XLU_LANES].T,
                          mxu_t(xa_ref[:, _XLU_LANES:])], axis=0)
    tb = mxu_t(xb_ref[...])
    out_ref[...] = jnp.concatenate([ta, tb], axis=1)


def _tc_transpose(table_t):
    eye = jnp.eye(_D, dtype=jnp.bfloat16)
    return pl.pallas_call(
        _transpose_body,
        grid=(_TGRID,),
        in_specs=[
            pl.BlockSpec((_D, _TBLK), lambda i: (0, i)),
            pl.BlockSpec((_D, _TBLK), lambda i: (0, i + _TGRID - 1)),
            pl.BlockSpec((_D, _D), lambda i: (0, 0)),
        ],
        out_specs=pl.BlockSpec((_TBLK, 2 * _D), lambda i: (i, 0)),
        out_shape=jax.ShapeDtypeStruct((_R, 2 * _D), jnp.float32),
    )(table_t, table_t, eye)


_BLK = 2048


def _mlp_body(g_ref, p_ref, s_ref, w1a_ref, w1s_ref, b1_ref, w2_ref, b2_ref,
              out_ref):
    parts = []
    for j in range(4):
        lo = g_ref[:, j * 2 * _D: j * 2 * _D + _D]
        hi = g_ref[:, j * 2 * _D + _D: (j + 1) * 2 * _D]
        pj = p_ref[:, j][:, None]
        parts.append(jnp.where(pj > 0.5, hi, lo))
    x = jnp.concatenate(parts, axis=1).astype(jnp.bfloat16)
    h = jnp.dot(x, w1a_ref[...].astype(jnp.bfloat16),
                preferred_element_type=jnp.float32)
    h = h + jnp.dot(s_ref[...], w1s_ref[...],
                    preferred_element_type=jnp.float32)
    h = jnp.maximum(h + b1_ref[...][None, :], 0.0)
    out_ref[...] = jnp.sum(h * w2_ref[...][None, :], axis=1) + b2_ref[...]


def _tc_mlp(g, p, s, w1a, w1s, b1, w2row, b2):
    grid = _B // _BLK
    return pl.pallas_call(
        _mlp_body,
        grid=(grid,),
        in_specs=[
            pl.BlockSpec((_BLK, 8 * _D), lambda i: (i, 0)),
            pl.BlockSpec((_BLK, 8), lambda i: (i, 0)),
            pl.BlockSpec((_BLK, 8), lambda i: (i, 0)),
            pl.BlockSpec((4 * _D, _H), lambda i: (0, 0)),
            pl.BlockSpec((8, _H), lambda i: (0, 0)),
            pl.BlockSpec((_H,), lambda i: (0,)),
            pl.BlockSpec((_H,), lambda i: (0,)),
            pl.BlockSpec((1,), lambda i: (0,)),
        ],
        out_specs=pl.BlockSpec((_BLK,), lambda i: (i,)),
        out_shape=jax.ShapeDtypeStruct((_B,), jnp.float32),
    )(g, p, s, w1a, w1s, b1, w2row, b2)


def kernel(target_item_ids, original_item_ids, left_item_ids, right_item_ids,
           position_indices, normalized_positions, session_lengths,
           prefix_scores, has_prefixes, item_embedding, W1, b1, W2, b2):
    ids = jnp.stack([target_item_ids, original_item_ids,
                     left_item_ids, right_item_ids], axis=1)
    half = ids >= _HOFF
    idx = jnp.where(half, ids - _HOFF, ids).reshape(_B4)
    parity = half.astype(jnp.int32)

    table2 = _tc_transpose(item_embedding.T)
    gathered = _sc_gather(table2, idx)
    g = gathered.reshape(_B, 8 * _D)

    p = jnp.concatenate([parity.astype(jnp.float32),
                         jnp.zeros((_B, 4), jnp.float32)], axis=1)
    zeros = jnp.zeros((_B,), jnp.float32)
    s = jnp.stack([position_indices, normalized_positions, session_lengths,
                   prefix_scores, has_prefixes, zeros, zeros, zeros], axis=1)

    w1a = W1[: 4 * _D]
    w1s = jnp.zeros((8, _H), jnp.float32).at[:5].set(W1[4 * _D:])
    w2row = W2.reshape(_H)
    return _tc_mlp(g, p, s, w1a, w1s, b1, w2row, b2)
